# baseline (speedup 1.0000x reference)
import dataclasses
import functools
import math

import jax
import jax.numpy as jnp
from jax import lax
from jax.experimental import pallas as pl
from jax.experimental.pallas import tpu as pltpu
from jax.experimental.pallas import tpu_sc as plsc

_LANES = 16
_NW = 32


def _sc_gather_affine(xf, ef, mul_t, bias_t):
    n = xf.shape[0]
    per_w = n // _NW
    e = mul_t.shape[0]
    mesh = plsc.VectorSubcoreMesh(core_axis_name="c", subcore_axis_name="s")
    cp = pltpu.CompilerParams()
    if "needs_layout_passes" in pltpu.CompilerParams.__dataclass_fields__:
        cp = dataclasses.replace(cp, needs_layout_passes=False)

    nch = 4
    ch = per_w // nch

    @functools.partial(
        pl.kernel,
        compiler_params=cp,
        out_type=jax.ShapeDtypeStruct((n,), jnp.float32),
        mesh=mesh,
        scratch_types=[
            pltpu.VMEM((ch,), jnp.int32),
            pltpu.VMEM((ch,), jnp.int32),
            pltpu.VMEM((ch,), jnp.float32),
            pltpu.VMEM((ch,), jnp.float32),
            pltpu.VMEM((ch,), jnp.float32),
            pltpu.VMEM((ch,), jnp.float32),
            pltpu.VMEM((e,), jnp.float32),
            pltpu.VMEM((e,), jnp.float32),
            pltpu.SemaphoreType.DMA,
            pltpu.SemaphoreType.DMA,
            pltpu.SemaphoreType.DMA,
            pltpu.SemaphoreType.DMA,
        ],
    )
    def k(x_hbm, e_hbm, mul_hbm, bias_hbm, out_hbm, idx0, idx1, xv0, xv1,
          ov0, ov1, mul_v, bias_v, si0, si1, so0, so1):
        wid = lax.axis_index("s") * 2 + lax.axis_index("c")
        base = wid * per_w
        pltpu.sync_copy(mul_hbm, mul_v)
        pltpu.sync_copy(bias_hbm, bias_v)
        idxb, xb, ob = (idx0, idx1), (xv0, xv1), (ov0, ov1)
        si, so = (si0, si1), (so0, so1)
        in_h, out_h = {}, {}

        def start_in(c):
            b = c & 1
            off = base + c * ch
            in_h[c] = (
                pltpu.async_copy(e_hbm.at[pl.ds(off, ch)], idxb[b], si[b]),
                pltpu.async_copy(x_hbm.at[pl.ds(off, ch)], xb[b], si[b]),
            )

        start_in(0)
        for c in range(nch):
            b = c & 1
            if c + 1 < nch:
                start_in(c + 1)
            in_h[c][0].wait()
            in_h[c][1].wait()
            if c >= 2:
                out_h[c - 2].wait()

            @pl.loop(0, ch, step=_LANES, unroll=4)
            def _(j, _b=b):
                iv = idxb[_b][pl.ds(j, _LANES)]
                mv = plsc.load_gather(mul_v, [iv])
                bv = plsc.load_gather(bias_v, [iv])
                ob[_b][pl.ds(j, _LANES)] = mv * xb[_b][pl.ds(j, _LANES)] + bv

            out_h[c] = pltpu.async_copy(
                ob[b], out_hbm.at[pl.ds(base + c * ch, ch)], so[b])

        out_h[nch - 2].wait()
        out_h[nch - 1].wait()

    return k(xf, ef, mul_t, bias_t)


def _tc_body(x_ref, c_ref, o_ref, *, bg, k):
    xv = x_ref[...]
    mean = c_ref[0:1, :]
    q = c_ref[1:2, :]
    coef = c_ref[2:3, :]
    xt = xv.T
    for s in range(bg):
        xcol = xt[:, s:s + 1]
        xb = jnp.broadcast_to(xcol, (128, k))
        d = xb - mean
        o_ref[s] = coef * jnp.exp2(d * d * q)


def _tc_gaussian(xx, consts, bg):
    n = xx.shape[0]
    k = consts.shape[1]
    g = n // 128
    return pl.pallas_call(
        functools.partial(_tc_body, bg=bg, k=k),
        grid=(g // bg,),
        in_specs=[
            pl.BlockSpec((bg, 128), lambda i: (i, 0)),
            pl.BlockSpec((8, k), lambda i: (0, 0)),
        ],
        out_specs=pl.BlockSpec((bg, 128, k), lambda i: (i, 0, 0)),
        out_shape=jax.ShapeDtypeStruct((g, 128, k), jnp.float32),
    )(xx.reshape(g, 128), consts)


def kernel(x, edge_type, means, stds, mul_weight, bias_weight):
    b, n, _ = x.shape
    k = means.shape[0]
    bnn = b * n * n

    xx = _sc_gather_affine(
        x.reshape(bnn),
        edge_type.reshape(bnn),
        mul_weight.reshape(-1),
        bias_weight.reshape(-1),
    )

    a = 1.0 / math.sqrt(2.0 * math.pi)
    log2e = math.log2(math.e)
    inv = 1.0 / (stds + 1e-6)
    consts = jnp.zeros((8, k), jnp.float32)
    consts = consts.at[0].set(means)
    consts = consts.at[1].set(-0.5 * log2e * inv * inv)
    consts = consts.at[2].set(a * inv)

    out = _tc_gaussian(xx, consts, bg=256)
    return out.reshape(b, n, n, k)


# speedup vs baseline: 1.0028x; 1.0028x over previous
import dataclasses
import functools
import math

import jax
import jax.numpy as jnp
from jax import lax
from jax.experimental import pallas as pl
from jax.experimental.pallas import tpu as pltpu
from jax.experimental.pallas import tpu_sc as plsc

_LANES = 16
_NW = 32


def _sc_gather_affine(xf, ef, mul_t, bias_t):
    n = xf.shape[0]
    per_w = n // _NW
    e = mul_t.shape[0]
    mesh = plsc.VectorSubcoreMesh(core_axis_name="c", subcore_axis_name="s")
    cp = pltpu.CompilerParams()
    if "needs_layout_passes" in pltpu.CompilerParams.__dataclass_fields__:
        cp = dataclasses.replace(cp, needs_layout_passes=False)

    @functools.partial(
        pl.kernel,
        compiler_params=cp,
        out_type=jax.ShapeDtypeStruct((n,), jnp.float32),
        mesh=mesh,
        scratch_types=[
            pltpu.VMEM((per_w,), jnp.int32),
            pltpu.VMEM((per_w,), jnp.float32),
            pltpu.VMEM((per_w,), jnp.float32),
            pltpu.VMEM((e,), jnp.float32),
            pltpu.VMEM((e,), jnp.float32),
        ],
    )
    def k(x_hbm, e_hbm, mul_hbm, bias_hbm, out_hbm, idx_v, x_v, out_v,
          mul_v, bias_v):
        wid = lax.axis_index("s") * 2 + lax.axis_index("c")
        base = wid * per_w
        pltpu.sync_copy(mul_hbm, mul_v)
        pltpu.sync_copy(bias_hbm, bias_v)
        pltpu.sync_copy(e_hbm.at[pl.ds(base, per_w)], idx_v)
        pltpu.sync_copy(x_hbm.at[pl.ds(base, per_w)], x_v)

        @pl.loop(0, per_w, step=_LANES, unroll=4)
        def _(j):
            iv = idx_v[pl.ds(j, _LANES)]
            mv = plsc.load_gather(mul_v, [iv])
            bv = plsc.load_gather(bias_v, [iv])
            out_v[pl.ds(j, _LANES)] = mv * x_v[pl.ds(j, _LANES)] + bv

        pltpu.sync_copy(out_v, out_hbm.at[pl.ds(base, per_w)])

    return k(xf, ef, mul_t, bias_t)


def _tc_body(x_ref, c_ref, o_ref, *, bg, k):
    xv = x_ref[...]
    mean = c_ref[0:1, :]
    q = c_ref[1:2, :]
    coef = c_ref[2:3, :]
    xt = xv.T
    for s in range(bg):
        xcol = xt[:, s:s + 1]
        xb = jnp.broadcast_to(xcol, (128, k))
        d = xb - mean
        o_ref[s] = coef * jnp.exp2(d * d * q)


def _tc_gaussian(xx, consts, bg):
    n = xx.shape[0]
    k = consts.shape[1]
    g = n // 128
    return pl.pallas_call(
        functools.partial(_tc_body, bg=bg, k=k),
        grid=(g // bg,),
        in_specs=[
            pl.BlockSpec((bg, 128), lambda i: (i, 0)),
            pl.BlockSpec((8, k), lambda i: (0, 0)),
        ],
        out_specs=pl.BlockSpec((bg, 128, k), lambda i: (i, 0, 0)),
        out_shape=jax.ShapeDtypeStruct((g, 128, k), jnp.float32),
    )(xx.reshape(g, 128), consts)


def kernel(x, edge_type, means, stds, mul_weight, bias_weight):
    b, n, _ = x.shape
    k = means.shape[0]
    bnn = b * n * n

    xx = _sc_gather_affine(
        x.reshape(bnn),
        edge_type.reshape(bnn),
        mul_weight.reshape(-1),
        bias_weight.reshape(-1),
    )

    a = 1.0 / math.sqrt(2.0 * math.pi)
    log2e = math.log2(math.e)
    inv = 1.0 / (stds + 1e-6)
    consts = jnp.zeros((8, k), jnp.float32)
    consts = consts.at[0].set(means)
    consts = consts.at[1].set(-0.5 * log2e * inv * inv)
    consts = consts.at[2].set(a * inv)

    out = _tc_gaussian(xx, consts, bg=256)
    return out.reshape(b, n, n, k)


# speedup vs baseline: 1.0339x; 1.0310x over previous
import dataclasses
import functools
import math

import jax
import jax.numpy as jnp
from jax import lax
from jax.experimental import pallas as pl
from jax.experimental.pallas import tpu as pltpu
from jax.experimental.pallas import tpu_sc as plsc

_LANES = 16
_NW = 32


def _sc_gather_affine(xf, ef, mul_t, bias_t):
    n = xf.shape[0]
    per_w = n // _NW
    e = mul_t.shape[0]
    mesh = plsc.VectorSubcoreMesh(core_axis_name="c", subcore_axis_name="s")
    cp = pltpu.CompilerParams()
    if "needs_layout_passes" in pltpu.CompilerParams.__dataclass_fields__:
        cp = dataclasses.replace(cp, needs_layout_passes=False)

    @functools.partial(
        pl.kernel,
        compiler_params=cp,
        out_type=jax.ShapeDtypeStruct((n,), jnp.float32),
        mesh=mesh,
        scratch_types=[
            pltpu.VMEM((per_w,), jnp.int32),
            pltpu.VMEM((per_w,), jnp.float32),
            pltpu.VMEM((per_w,), jnp.float32),
            pltpu.VMEM((e,), jnp.float32),
            pltpu.VMEM((e,), jnp.float32),
        ],
    )
    def k(x_hbm, e_hbm, mul_hbm, bias_hbm, out_hbm, idx_v, x_v, out_v,
          mul_v, bias_v):
        wid = lax.axis_index("s") * 2 + lax.axis_index("c")
        base = wid * per_w
        pltpu.sync_copy(mul_hbm, mul_v)
        pltpu.sync_copy(bias_hbm, bias_v)
        pltpu.sync_copy(e_hbm.at[pl.ds(base, per_w)], idx_v)
        pltpu.sync_copy(x_hbm.at[pl.ds(base, per_w)], x_v)

        @pl.loop(0, per_w, step=_LANES)
        def _(j):
            iv = idx_v[pl.ds(j, _LANES)]
            mv = plsc.load_gather(mul_v, [iv])
            bv = plsc.load_gather(bias_v, [iv])
            out_v[pl.ds(j, _LANES)] = mv * x_v[pl.ds(j, _LANES)] + bv

        pltpu.sync_copy(out_v, out_hbm.at[pl.ds(base, per_w)])

    return k(xf, ef, mul_t, bias_t)


def _tc_body(x_ref, c_ref, o_ref, *, bg, k):
    xv = x_ref[...]
    mean = c_ref[0:1, :]
    q = c_ref[1:2, :]
    coef = c_ref[2:3, :]
    xt = xv.T
    for s in range(bg):
        xcol = xt[:, s:s + 1]
        xb = jnp.broadcast_to(xcol, (128, k))
        d = xb - mean
        o_ref[s] = coef * jnp.exp2(d * d * q)


def _tc_gaussian(xx, consts, bg):
    n = xx.shape[0]
    k = consts.shape[1]
    g = n // 128
    return pl.pallas_call(
        functools.partial(_tc_body, bg=bg, k=k),
        grid=(g // bg,),
        in_specs=[
            pl.BlockSpec((bg, 128), lambda i: (i, 0)),
            pl.BlockSpec((8, k), lambda i: (0, 0)),
        ],
        out_specs=pl.BlockSpec((bg, 128, k), lambda i: (i, 0, 0)),
        out_shape=jax.ShapeDtypeStruct((g, 128, k), jnp.float32),
    )(xx.reshape(g, 128), consts)


def kernel(x, edge_type, means, stds, mul_weight, bias_weight):
    b, n, _ = x.shape
    k = means.shape[0]
    bnn = b * n * n

    xx = _sc_gather_affine(
        x.reshape(bnn),
        edge_type.reshape(bnn),
        mul_weight.reshape(-1),
        bias_weight.reshape(-1),
    )

    a = 1.0 / math.sqrt(2.0 * math.pi)
    log2e = math.log2(math.e)
    inv = 1.0 / (stds + 1e-6)
    consts = jnp.zeros((8, k), jnp.float32)
    consts = consts.at[0].set(means)
    consts = consts.at[1].set(-0.5 * log2e * inv * inv)
    consts = consts.at[2].set(a * inv)

    out = _tc_gaussian(xx, consts, bg=256)
    return out.reshape(b, n, n, k)


# speedup vs baseline: 1.0347x; 1.0007x over previous
import dataclasses
import functools
import math

import jax
import jax.numpy as jnp
from jax import lax
from jax.experimental import pallas as pl
from jax.experimental.pallas import tpu as pltpu
from jax.experimental.pallas import tpu_sc as plsc

_LANES = 16
_NW = 32


def _sc_gather_affine(xf, ef, mul_t, bias_t):
    n = xf.shape[0]
    per_w = n // _NW
    e = mul_t.shape[0]
    mesh = plsc.VectorSubcoreMesh(core_axis_name="c", subcore_axis_name="s")
    cp = pltpu.CompilerParams()
    if "needs_layout_passes" in pltpu.CompilerParams.__dataclass_fields__:
        cp = dataclasses.replace(cp, needs_layout_passes=False)

    @functools.partial(
        pl.kernel,
        compiler_params=cp,
        out_type=jax.ShapeDtypeStruct((n,), jnp.float32),
        mesh=mesh,
        scratch_types=[
            pltpu.VMEM((per_w,), jnp.int32),
            pltpu.VMEM((per_w,), jnp.float32),
            pltpu.VMEM((per_w,), jnp.float32),
            pltpu.VMEM((e,), jnp.float32),
            pltpu.VMEM((e,), jnp.float32),
        ],
    )
    def k(x_hbm, e_hbm, mul_hbm, bias_hbm, out_hbm, idx_v, x_v, out_v,
          mul_v, bias_v):
        wid = lax.axis_index("s") * 2 + lax.axis_index("c")
        base = wid * per_w
        pltpu.sync_copy(mul_hbm, mul_v)
        pltpu.sync_copy(bias_hbm, bias_v)
        pltpu.sync_copy(e_hbm.at[pl.ds(base, per_w)], idx_v)
        pltpu.sync_copy(x_hbm.at[pl.ds(base, per_w)], x_v)

        @pl.loop(0, per_w, step=_LANES)
        def _(j):
            iv = idx_v[pl.ds(j, _LANES)]
            mv = plsc.load_gather(mul_v, [iv])
            bv = plsc.load_gather(bias_v, [iv])
            out_v[pl.ds(j, _LANES)] = mv * x_v[pl.ds(j, _LANES)] + bv

        pltpu.sync_copy(out_v, out_hbm.at[pl.ds(base, per_w)])

    return k(xf, ef, mul_t, bias_t)


def _tc_body(x_ref, c_ref, o_ref, *, bg, k):
    xv = x_ref[...]
    mean = c_ref[0:1, :]
    q = c_ref[1:2, :]
    coef = c_ref[2:3, :]
    xt = xv.T
    for s in range(bg):
        xcol = xt[:, s:s + 1]
        xb = jnp.broadcast_to(xcol, (128, k))
        d = xb - mean
        o_ref[s] = coef * jnp.exp2(d * d * q)


def _tc_gaussian(xx, consts, bg):
    n = xx.shape[0]
    k = consts.shape[1]
    g = n // 128
    return pl.pallas_call(
        functools.partial(_tc_body, bg=bg, k=k),
        grid=(g // bg,),
        in_specs=[
            pl.BlockSpec((bg, 128), lambda i: (i, 0)),
            pl.BlockSpec((8, k), lambda i: (0, 0)),
        ],
        out_specs=pl.BlockSpec((bg, 128, k), lambda i: (i, 0, 0)),
        out_shape=jax.ShapeDtypeStruct((g, 128, k), jnp.float32),
        compiler_params=pltpu.CompilerParams(
            dimension_semantics=("parallel",)),
    )(xx.reshape(g, 128), consts)


def kernel(x, edge_type, means, stds, mul_weight, bias_weight):
    b, n, _ = x.shape
    k = means.shape[0]
    bnn = b * n * n

    xx = _sc_gather_affine(
        x.reshape(bnn),
        edge_type.reshape(bnn),
        mul_weight.reshape(-1),
        bias_weight.reshape(-1),
    )

    a = 1.0 / math.sqrt(2.0 * math.pi)
    log2e = math.log2(math.e)
    inv = 1.0 / (stds + 1e-6)
    consts = jnp.zeros((8, k), jnp.float32)
    consts = consts.at[0].set(means)
    consts = consts.at[1].set(-0.5 * log2e * inv * inv)
    consts = consts.at[2].set(a * inv)

    out = _tc_gaussian(xx, consts, bg=256)
    return out.reshape(b, n, n, k)
